# Initial kernel scaffold; baseline (speedup 1.0000x reference)
import jax
import jax.numpy as jnp
from jax.experimental import pallas as pl


def kernel(x, gamma):
    raise NotImplementedError("write your pallas kernel here")


# speedup vs baseline: 1.2743x; 1.2743x over previous
import jax
import jax.numpy as jnp
from jax.experimental import pallas as pl
from jax.experimental.pallas import tpu as pltpu

C = 64
BN = 12288


def _gram_kernel(x_ref, aff_ref, acc_ref):
    n = pl.program_id(1)

    @pl.when(n == 0)
    def _():
        acc_ref[...] = jnp.zeros_like(acc_ref)

    xb = x_ref[0]
    acc_ref[...] += jax.lax.dot_general(
        xb, xb, (((1,), (1,)), ((), ())), preferred_element_type=jnp.float32)

    @pl.when(n == pl.num_programs(1) - 1)
    def _():
        g = acc_ref[...]
        m3 = jnp.dot(g, g, preferred_element_type=jnp.float32)
        aff_ref[0] = jax.nn.sigmoid(m3)


def _weights_kernel(aff_ref, x_ref, gamma_ref, o_ref):
    w = jnp.dot(aff_ref[0], x_ref[0], preferred_element_type=jnp.float32)
    o_ref[0] = gamma_ref[0] * w + x_ref[0]


def kernel(x, gamma):
    B, W, D, H, Cx = x.shape
    N = W * D * H
    k = x.reshape(B, Cx, N)
    nb = N // BN

    aff = pl.pallas_call(
        _gram_kernel,
        grid=(B, nb),
        in_specs=[pl.BlockSpec((1, C, BN), lambda b, n: (b, 0, n))],
        out_specs=pl.BlockSpec((1, C, C), lambda b, n: (b, 0, 0)),
        out_shape=jax.ShapeDtypeStruct((B, C, C), jnp.float32),
        scratch_shapes=[pltpu.VMEM((C, C), jnp.float32)],
        compiler_params=pltpu.CompilerParams(
            dimension_semantics=("parallel", "arbitrary")),
    )(k)

    out = pl.pallas_call(
        _weights_kernel,
        grid=(B, nb),
        in_specs=[
            pl.BlockSpec((1, C, C), lambda b, n: (b, 0, 0)),
            pl.BlockSpec((1, C, BN), lambda b, n: (b, 0, n)),
            pl.BlockSpec(memory_space=pltpu.SMEM),
        ],
        out_specs=pl.BlockSpec((1, C, BN), lambda b, n: (b, 0, n)),
        out_shape=jax.ShapeDtypeStruct((B, C, N), jnp.float32),
        compiler_params=pltpu.CompilerParams(
            dimension_semantics=("parallel", "arbitrary")),
    )(aff, k, gamma.reshape(1))

    return out.reshape(B, W, D, H, Cx)


# speedup vs baseline: 1.3403x; 1.0518x over previous
import jax
import jax.numpy as jnp
from jax.experimental import pallas as pl
from jax.experimental.pallas import tpu as pltpu

C = 64
BN = 12288
NC = 9


def _fused_kernel(gamma_ref, x_hbm, o_hbm, xbuf, obuf, in_sems, out_sems):
    b = pl.program_id(0)

    in_cps = [
        pltpu.make_async_copy(
            x_hbm.at[b, :, pl.ds(i * BN, BN)], xbuf.at[i], in_sems.at[i])
        for i in range(NC)
    ]
    for cp in in_cps:
        cp.start()

    g = None
    for i in range(NC):
        in_cps[i].wait()
        xb = xbuf[i]
        d = jax.lax.dot_general(
            xb, xb, (((1,), (1,)), ((), ())),
            preferred_element_type=jnp.float32)
        g = d if g is None else g + d

    m3 = jnp.dot(g, g, preferred_element_type=jnp.float32)
    aff = jax.nn.sigmoid(m3)
    gamma = gamma_ref[0]

    out_cps = []
    for i in range(NC):
        s = i % 2
        if i >= 2:
            out_cps[i - 2].wait()
        w = jnp.dot(aff, xbuf[i], preferred_element_type=jnp.float32)
        obuf[s] = gamma * w + xbuf[i]
        cp = pltpu.make_async_copy(
            obuf.at[s], o_hbm.at[b, :, pl.ds(i * BN, BN)], out_sems.at[s])
        cp.start()
        out_cps.append(cp)
    out_cps[-2].wait()
    out_cps[-1].wait()


def kernel(x, gamma):
    B, W, D, H, Cx = x.shape
    N = W * D * H
    k = x.reshape(B, Cx, N)

    out = pl.pallas_call(
        _fused_kernel,
        grid=(B,),
        in_specs=[
            pl.BlockSpec(memory_space=pltpu.SMEM),
            pl.BlockSpec(memory_space=pl.ANY),
        ],
        out_specs=pl.BlockSpec(memory_space=pl.ANY),
        out_shape=jax.ShapeDtypeStruct((B, C, N), jnp.float32),
        scratch_shapes=[
            pltpu.VMEM((NC, C, BN), jnp.float32),
            pltpu.VMEM((2, C, BN), jnp.float32),
            pltpu.SemaphoreType.DMA((NC,)),
            pltpu.SemaphoreType.DMA((2,)),
        ],
        compiler_params=pltpu.CompilerParams(
            dimension_semantics=("parallel",),
            vmem_limit_bytes=50 * 1024 * 1024),
    )(gamma.reshape(1), k)

    return out.reshape(B, W, D, H, Cx)
